# baseline (speedup 1.0000x reference)
import functools
import jax
import jax.numpy as jnp
from jax import lax
from jax.experimental import pallas as pl
from jax.experimental.pallas import tpu as pltpu
from jax.experimental.pallas import tpu_sc as plsc

D_MODEL = 768
N_EXP = 8
TOPK = 2
HID = 2048
SEQ = 2048
BT = 256
NBLK = (TOPK * SEQ) // BT + N_EXP
GP = NBLK * BT
NW = 32
CT = 64
_CS = 256


def _router_body(logits_ref, gates_ref, pos_ref, be_ref, bv_ref):
    logits = logits_ref[...]
    ei = jax.lax.broadcasted_iota(jnp.int32, (SEQ, N_EXP), 1)
    m0 = jnp.max(logits, axis=1, keepdims=True)
    e0 = jnp.min(jnp.where(logits == m0, ei, N_EXP), axis=1, keepdims=True)
    l2 = jnp.where(ei == e0, -jnp.inf, logits)
    m1 = jnp.max(l2, axis=1, keepdims=True)
    e1 = jnp.min(jnp.where(l2 == m1, ei, N_EXP), axis=1, keepdims=True)
    gates_ref[:, 0:1] = jax.nn.sigmoid(m0 - m1)
    gates_ref[:, 1:2] = jax.nn.sigmoid(m1 - m0)

    onehot0 = (ei == e0).astype(jnp.float32)
    onehot1 = (ei == e1).astype(jnp.float32)
    combined = (onehot0 + onehot1).astype(jnp.bfloat16)

    ri = jax.lax.broadcasted_iota(jnp.int32, (_CS, _CS), 0)
    ci = jax.lax.broadcasted_iota(jnp.int32, (_CS, _CS), 1)
    ltri = (ci < ri).astype(jnp.bfloat16)
    carry = jnp.zeros((1, N_EXP), jnp.float32)
    rank0_parts, rank1_parts = [], []
    for c in range(SEQ // _CS):
        sl = slice(c * _CS, (c + 1) * _CS)
        chunk = combined[sl, :]
        excl = jax.lax.dot_general(ltri, chunk, (((1,), (0,)), ((), ())),
                                   preferred_element_type=jnp.float32) + carry
        chunk_f = chunk.astype(jnp.float32)
        carry = excl[_CS - 1:_CS, :] + chunk_f[_CS - 1:_CS, :]
        rank0_parts.append(jnp.sum(excl * onehot0[sl, :], axis=1, keepdims=True))
        rank1_parts.append(jnp.sum((excl + onehot0[sl, :]) * onehot1[sl, :],
                                   axis=1, keepdims=True))
    counts = carry
    rank0 = jnp.concatenate(rank0_parts, axis=0)
    rank1 = jnp.concatenate(rank1_parts, axis=0)

    counts_i = counts.astype(jnp.int32)
    nb = (counts_i + BT - 1) // BT
    nb_f = nb.astype(jnp.bfloat16)
    u8r = jax.lax.broadcasted_iota(jnp.int32, (N_EXP, N_EXP), 0)
    u8c = jax.lax.broadcasted_iota(jnp.int32, (N_EXP, N_EXP), 1)
    utri = (u8r <= u8c).astype(jnp.bfloat16)
    nb_cum = jax.lax.dot_general(nb_f, utri, (((1,), (0,)), ((), ())),
                                 preferred_element_type=jnp.float32)
    pad_start = (BT * (nb_cum - nb.astype(jnp.float32)))

    p0 = jnp.sum(pad_start * onehot0, axis=1, keepdims=True) + rank0
    p1 = jnp.sum(pad_start * onehot1, axis=1, keepdims=True) + rank1
    pos_ref[:, 0:1] = p0.astype(jnp.int32)
    pos_ref[:, 1:2] = p1.astype(jnp.int32)

    gi = jax.lax.broadcasted_iota(jnp.int32, (N_EXP, NBLK), 1).astype(jnp.float32)
    nb_cum_t = jnp.transpose(nb_cum)
    bige = jnp.sum((gi >= nb_cum_t).astype(jnp.int32), axis=0, keepdims=True)
    be_ref[...] = jnp.minimum(bige, N_EXP - 1)
    nb_tot = nb_cum[:, N_EXP - 1:N_EXP]
    gi1 = jax.lax.broadcasted_iota(jnp.int32, (1, NBLK), 1).astype(jnp.float32)
    bv_ref[...] = (gi1 < nb_tot).astype(jnp.int32)


def _sc_gather(x2, idx):
    mesh = plsc.VectorSubcoreMesh(core_axis_name="c", subcore_axis_name="s")
    b_per_w = GP // NW
    CH = 64

    @functools.partial(
        pl.kernel, mesh=mesh,
        out_type=jax.ShapeDtypeStruct((GP, D_MODEL), jnp.float32),
        scratch_types=[
            pltpu.VMEM((CH,), jnp.int32),
            pltpu.VMEM((CH,), jnp.int32),
            pltpu.VMEM((CH, D_MODEL), jnp.float32),
            pltpu.VMEM((CH, D_MODEL), jnp.float32),
            pltpu.SemaphoreType.DMA,
            pltpu.SemaphoreType.DMA,
        ],
    )
    def k(x_hbm, idx_hbm, out_hbm, iv0, iv1, rv0, rv1, s0, s1):
        wid = lax.axis_index("s") * 2 + lax.axis_index("c")
        base = wid * b_per_w
        pltpu.sync_copy(idx_hbm.at[pl.ds(base, CH)], iv0)
        g0 = pltpu.async_copy(x_hbm.at[iv0], rv0, s0)
        pltpu.sync_copy(idx_hbm.at[pl.ds(base + CH, CH)], iv1)
        g1 = pltpu.async_copy(x_hbm.at[iv1], rv1, s1)
        g0.wait()
        pltpu.sync_copy(rv0, out_hbm.at[pl.ds(base, CH)])
        pltpu.sync_copy(idx_hbm.at[pl.ds(base + 2 * CH, CH)], iv0)
        g2 = pltpu.async_copy(x_hbm.at[iv0], rv0, s0)
        g1.wait()
        pltpu.sync_copy(rv1, out_hbm.at[pl.ds(base + CH, CH)])
        g2.wait()
        pltpu.sync_copy(rv0, out_hbm.at[pl.ds(base + 2 * CH, CH)])

    return k(x2, idx)


def _sc_combine(slots, pos2, g0e, g1e):
    mesh = plsc.VectorSubcoreMesh(core_axis_name="c", subcore_axis_name="s")
    t_per_w = SEQ // NW

    @functools.partial(
        pl.kernel, mesh=mesh,
        out_type=jax.ShapeDtypeStruct((SEQ, D_MODEL), jnp.float32),
        scratch_types=[
            pltpu.VMEM((CT,), jnp.int32),
            pltpu.VMEM((CT,), jnp.int32),
            pltpu.VMEM((CT, D_MODEL), jnp.float32),
            pltpu.VMEM((CT, D_MODEL), jnp.float32),
            pltpu.VMEM((CT, 16), jnp.float32),
            pltpu.VMEM((CT, 16), jnp.float32),
            pltpu.SemaphoreType.DMA,
            pltpu.SemaphoreType.DMA,
        ],
    )
    def k(slots_hbm, pos_hbm, g0_hbm, g1_hbm, y_hbm,
          i0_v, i1_v, r0_v, r1_v, gv0, gv1, s0, s1):
        wid = lax.axis_index("s") * 2 + lax.axis_index("c")
        base = wid * t_per_w
        for c in range(t_per_w // CT):
            off = base + c * CT
            pltpu.sync_copy(pos_hbm.at[0, pl.ds(off, CT)], i0_v)
            pltpu.sync_copy(pos_hbm.at[1, pl.ds(off, CT)], i1_v)
            cp0 = pltpu.async_copy(slots_hbm.at[i0_v], r0_v, s0)
            cp1 = pltpu.async_copy(slots_hbm.at[i1_v], r1_v, s1)
            pltpu.sync_copy(g0_hbm.at[pl.ds(off, CT)], gv0)
            pltpu.sync_copy(g1_hbm.at[pl.ds(off, CT)], gv1)
            cp0.wait()
            cp1.wait()

            @pl.loop(0, CT)
            def _(i):
                a = gv0[i, :]
                b = gv1[i, :]
                for j in range(D_MODEL // 16):
                    sl = pl.ds(j * 16, 16)
                    r0_v[i, sl] = r0_v[i, sl] * a + r1_v[i, sl] * b

            pltpu.sync_copy(r0_v, y_hbm.at[pl.ds(off, CT)])

    return k(slots, pos2, g0e, g1e)


def _ffn_body(be_ref, bv_ref, xs_ref, w1_ref, w2_ref, w3_ref,
              b1_ref, b2_ref, b3_ref, out_ref):
    g = pl.program_id(0)

    @pl.when(bv_ref[g] == 1)
    def _compute():
        xgb = xs_ref[...].astype(jnp.bfloat16)
        w1 = w1_ref[0].astype(jnp.bfloat16)
        w2 = w2_ref[0].astype(jnp.bfloat16)
        w3 = w3_ref[0].astype(jnp.bfloat16)
        h1 = jax.lax.dot_general(xgb, w1, (((1,), (0,)), ((), ())),
                                 preferred_element_type=jnp.float32) + b1_ref[0]
        h2 = jax.lax.dot_general(xgb, w2, (((1,), (0,)), ((), ())),
                                 preferred_element_type=jnp.float32) + b2_ref[0]
        hs = (h1 * (h2 * jax.nn.sigmoid(h2))).astype(jnp.bfloat16)
        out_ref[...] = jax.lax.dot_general(
            hs, w3, (((1,), (0,)), ((), ())),
            preferred_element_type=jnp.float32) + b3_ref[0]


def kernel(x, router_w, router_b, fc1_w, fc1_b, fc2_w, fc2_b, fc3_w, fc3_b):
    x2 = x.reshape(SEQ, D_MODEL)

    logits = (jnp.einsum('btd,de->bte', x, router_w) + router_b).reshape(SEQ, N_EXP)

    gates, pos, be, bv = pl.pallas_call(
        _router_body,
        out_shape=(jax.ShapeDtypeStruct((SEQ, TOPK), jnp.float32),
                   jax.ShapeDtypeStruct((SEQ, TOPK), jnp.int32),
                   jax.ShapeDtypeStruct((1, NBLK), jnp.int32),
                   jax.ShapeDtypeStruct((1, NBLK), jnp.int32)),
    )(logits)

    pos_slot = pos.reshape(TOPK * SEQ)
    tokK = jnp.arange(TOPK * SEQ, dtype=jnp.int32) // TOPK
    gather_tok = jnp.zeros((GP,), jnp.int32).at[pos_slot].set(tokK)
    xs = _sc_gather(x2, gather_tok)

    grid_spec = pltpu.PrefetchScalarGridSpec(
        num_scalar_prefetch=2,
        grid=(NBLK,),
        in_specs=[
            pl.BlockSpec((BT, D_MODEL), lambda g, be, bv: (g, 0)),
            pl.BlockSpec((1, D_MODEL, HID), lambda g, be, bv: (be[g], 0, 0)),
            pl.BlockSpec((1, D_MODEL, HID), lambda g, be, bv: (be[g], 0, 0)),
            pl.BlockSpec((1, HID, D_MODEL), lambda g, be, bv: (be[g], 0, 0)),
            pl.BlockSpec((1, 1, HID), lambda g, be, bv: (be[g], 0, 0)),
            pl.BlockSpec((1, 1, HID), lambda g, be, bv: (be[g], 0, 0)),
            pl.BlockSpec((1, 1, D_MODEL), lambda g, be, bv: (be[g], 0, 0)),
        ],
        out_specs=pl.BlockSpec((BT, D_MODEL), lambda g, be, bv: (g, 0)),
    )
    slots = pl.pallas_call(
        _ffn_body,
        grid_spec=grid_spec,
        out_shape=jax.ShapeDtypeStruct((GP, D_MODEL), jnp.float32),
    )(be.reshape(NBLK), bv.reshape(NBLK), xs,
      fc1_w, fc2_w, fc3_w,
      fc1_b.reshape(N_EXP, 1, HID), fc2_b.reshape(N_EXP, 1, HID),
      fc3_b.reshape(N_EXP, 1, D_MODEL))

    g0e = jnp.broadcast_to(gates[:, 0:1], (SEQ, 16))
    g1e = jnp.broadcast_to(gates[:, 1:2], (SEQ, 16))
    y = _sc_combine(slots, pos.T, g0e, g1e)
    return y.reshape(1, SEQ, D_MODEL)


# speedup vs baseline: 1.5619x; 1.5619x over previous
import functools
import jax
import jax.numpy as jnp
from jax import lax
from jax.experimental import pallas as pl
from jax.experimental.pallas import tpu as pltpu
from jax.experimental.pallas import tpu_sc as plsc

D_MODEL = 768
N_EXP = 8
TOPK = 2
HID = 2048
SEQ = 2048
BT = 256
NBLK = (TOPK * SEQ) // BT + N_EXP
GP = NBLK * BT
NW = 32
CT = 64
_CS = 256


def _router_body(logits_ref, gates_ref, pos_ref, be_ref, bv_ref):
    logits = logits_ref[...]
    ei = jax.lax.broadcasted_iota(jnp.int32, (SEQ, N_EXP), 1)
    m0 = jnp.max(logits, axis=1, keepdims=True)
    e0 = jnp.min(jnp.where(logits == m0, ei, N_EXP), axis=1, keepdims=True)
    l2 = jnp.where(ei == e0, -jnp.inf, logits)
    m1 = jnp.max(l2, axis=1, keepdims=True)
    e1 = jnp.min(jnp.where(l2 == m1, ei, N_EXP), axis=1, keepdims=True)
    gates_ref[:, 0:1] = jax.nn.sigmoid(m0 - m1)
    gates_ref[:, 1:2] = jax.nn.sigmoid(m1 - m0)

    onehot0 = (ei == e0).astype(jnp.float32)
    onehot1 = (ei == e1).astype(jnp.float32)
    combined = (onehot0 + onehot1).astype(jnp.bfloat16)

    ri = jax.lax.broadcasted_iota(jnp.int32, (_CS, _CS), 0)
    ci = jax.lax.broadcasted_iota(jnp.int32, (_CS, _CS), 1)
    ltri = (ci < ri).astype(jnp.bfloat16)
    carry = jnp.zeros((1, N_EXP), jnp.float32)
    rank0_parts, rank1_parts = [], []
    for c in range(SEQ // _CS):
        sl = slice(c * _CS, (c + 1) * _CS)
        chunk = combined[sl, :]
        excl = jax.lax.dot_general(ltri, chunk, (((1,), (0,)), ((), ())),
                                   preferred_element_type=jnp.float32) + carry
        chunk_f = chunk.astype(jnp.float32)
        carry = excl[_CS - 1:_CS, :] + chunk_f[_CS - 1:_CS, :]
        rank0_parts.append(jnp.sum(excl * onehot0[sl, :], axis=1, keepdims=True))
        rank1_parts.append(jnp.sum((excl + onehot0[sl, :]) * onehot1[sl, :],
                                   axis=1, keepdims=True))
    counts = carry
    rank0 = jnp.concatenate(rank0_parts, axis=0)
    rank1 = jnp.concatenate(rank1_parts, axis=0)

    counts_i = counts.astype(jnp.int32)
    nb = (counts_i + BT - 1) // BT
    nb_f = nb.astype(jnp.bfloat16)
    u8r = jax.lax.broadcasted_iota(jnp.int32, (N_EXP, N_EXP), 0)
    u8c = jax.lax.broadcasted_iota(jnp.int32, (N_EXP, N_EXP), 1)
    utri = (u8r <= u8c).astype(jnp.bfloat16)
    nb_cum = jax.lax.dot_general(nb_f, utri, (((1,), (0,)), ((), ())),
                                 preferred_element_type=jnp.float32)
    pad_start = (BT * (nb_cum - nb.astype(jnp.float32)))

    p0 = jnp.sum(pad_start * onehot0, axis=1, keepdims=True) + rank0
    p1 = jnp.sum(pad_start * onehot1, axis=1, keepdims=True) + rank1
    pos_ref[:, 0:1] = p0.astype(jnp.int32)
    pos_ref[:, 1:2] = p1.astype(jnp.int32)

    gi = jax.lax.broadcasted_iota(jnp.int32, (N_EXP, NBLK), 1).astype(jnp.float32)
    nb_cum_t = jnp.transpose(nb_cum)
    bige = jnp.sum((gi >= nb_cum_t).astype(jnp.int32), axis=0, keepdims=True)
    be_ref[...] = jnp.minimum(bige, N_EXP - 1)
    nb_tot = nb_cum[:, N_EXP - 1:N_EXP]
    gi1 = jax.lax.broadcasted_iota(jnp.int32, (1, NBLK), 1).astype(jnp.float32)
    bv_ref[...] = (gi1 < nb_tot).astype(jnp.int32)


def _sc_gather(x2, idx):
    mesh = plsc.VectorSubcoreMesh(core_axis_name="c", subcore_axis_name="s")
    b_per_w = GP // NW
    CH = 64

    @functools.partial(
        pl.kernel, mesh=mesh,
        out_type=jax.ShapeDtypeStruct((GP, D_MODEL), jnp.float32),
        scratch_types=[
            pltpu.VMEM((CH,), jnp.int32),
            pltpu.VMEM((CH,), jnp.int32),
            pltpu.VMEM((CH, D_MODEL), jnp.float32),
            pltpu.VMEM((CH, D_MODEL), jnp.float32),
            pltpu.SemaphoreType.DMA,
            pltpu.SemaphoreType.DMA,
        ],
    )
    def k(x_hbm, idx_hbm, out_hbm, iv0, iv1, rv0, rv1, s0, s1):
        wid = lax.axis_index("s") * 2 + lax.axis_index("c")
        base = wid * b_per_w
        pltpu.sync_copy(idx_hbm.at[pl.ds(base, CH)], iv0)
        g0 = pltpu.async_copy(x_hbm.at[iv0], rv0, s0)
        pltpu.sync_copy(idx_hbm.at[pl.ds(base + CH, CH)], iv1)
        g1 = pltpu.async_copy(x_hbm.at[iv1], rv1, s1)
        g0.wait()
        pltpu.sync_copy(rv0, out_hbm.at[pl.ds(base, CH)])
        pltpu.sync_copy(idx_hbm.at[pl.ds(base + 2 * CH, CH)], iv0)
        g2 = pltpu.async_copy(x_hbm.at[iv0], rv0, s0)
        g1.wait()
        pltpu.sync_copy(rv1, out_hbm.at[pl.ds(base + CH, CH)])
        g2.wait()
        pltpu.sync_copy(rv0, out_hbm.at[pl.ds(base + 2 * CH, CH)])

    return k(x2, idx)


def _sc_combine(slots, pos2, g0e, g1e):
    mesh = plsc.VectorSubcoreMesh(core_axis_name="c", subcore_axis_name="s")
    t_per_w = SEQ // NW

    @functools.partial(
        pl.kernel, mesh=mesh,
        out_type=jax.ShapeDtypeStruct((SEQ, D_MODEL), jnp.float32),
        scratch_types=[
            pltpu.VMEM((CT,), jnp.int32),
            pltpu.VMEM((CT,), jnp.int32),
            pltpu.VMEM((CT, D_MODEL), jnp.float32),
            pltpu.VMEM((CT, D_MODEL), jnp.float32),
            pltpu.VMEM((CT, 16), jnp.float32),
            pltpu.VMEM((CT, 16), jnp.float32),
            pltpu.SemaphoreType.DMA,
            pltpu.SemaphoreType.DMA,
        ],
    )
    def k(slots_hbm, pos_hbm, g0_hbm, g1_hbm, y_hbm,
          i0_v, i1_v, r0_v, r1_v, gv0, gv1, s0, s1):
        wid = lax.axis_index("s") * 2 + lax.axis_index("c")
        base = wid * t_per_w
        for c in range(t_per_w // CT):
            off = base + c * CT
            pltpu.sync_copy(pos_hbm.at[0, pl.ds(off, CT)], i0_v)
            pltpu.sync_copy(pos_hbm.at[1, pl.ds(off, CT)], i1_v)
            cp0 = pltpu.async_copy(slots_hbm.at[i0_v], r0_v, s0)
            cp1 = pltpu.async_copy(slots_hbm.at[i1_v], r1_v, s1)
            pltpu.sync_copy(g0_hbm.at[pl.ds(off, CT)], gv0)
            pltpu.sync_copy(g1_hbm.at[pl.ds(off, CT)], gv1)
            cp0.wait()
            cp1.wait()

            @pl.loop(0, CT)
            def _(i):
                a = gv0[i, :]
                b = gv1[i, :]
                for j in range(D_MODEL // 16):
                    sl = pl.ds(j * 16, 16)
                    r0_v[i, sl] = r0_v[i, sl] * a + r1_v[i, sl] * b

            pltpu.sync_copy(r0_v, y_hbm.at[pl.ds(off, CT)])

    return k(slots, pos2, g0e, g1e)


def _ffn_body(be_ref, bv_ref, xs_ref, w1_ref, w2_ref, w3_ref,
              b1_ref, b2_ref, b3_ref, out_ref):
    g = pl.program_id(0)

    @pl.when(bv_ref[g] == 1)
    def _compute():
        xgb = xs_ref[...].astype(jnp.bfloat16)
        w1 = w1_ref[0].astype(jnp.bfloat16)
        w2 = w2_ref[0].astype(jnp.bfloat16)
        w3 = w3_ref[0].astype(jnp.bfloat16)
        h1 = jax.lax.dot_general(xgb, w1, (((1,), (0,)), ((), ())),
                                 preferred_element_type=jnp.float32) + b1_ref[0]
        h2 = jax.lax.dot_general(xgb, w2, (((1,), (0,)), ((), ())),
                                 preferred_element_type=jnp.float32) + b2_ref[0]
        hs = (h1 * (h2 * jax.nn.sigmoid(h2))).astype(jnp.bfloat16)
        out_ref[...] = jax.lax.dot_general(
            hs, w3, (((1,), (0,)), ((), ())),
            preferred_element_type=jnp.float32) + b3_ref[0]


def kernel(x, router_w, router_b, fc1_w, fc1_b, fc2_w, fc2_b, fc3_w, fc3_b):
    x2 = x.reshape(SEQ, D_MODEL)

    logits = (jnp.einsum('btd,de->bte', x, router_w) + router_b).reshape(SEQ, N_EXP)

    gates, pos, be, bv = pl.pallas_call(
        _router_body,
        out_shape=(jax.ShapeDtypeStruct((SEQ, TOPK), jnp.float32),
                   jax.ShapeDtypeStruct((SEQ, TOPK), jnp.int32),
                   jax.ShapeDtypeStruct((1, NBLK), jnp.int32),
                   jax.ShapeDtypeStruct((1, NBLK), jnp.int32)),
    )(logits)

    pos_slot = pos.reshape(TOPK * SEQ)
    tokK = jnp.arange(TOPK * SEQ, dtype=jnp.int32) // TOPK
    pad_fill = (jnp.arange(GP, dtype=jnp.int32) % SEQ)
    gather_tok = pad_fill.at[pos_slot].set(tokK)
    xs = _sc_gather(x2, gather_tok)

    grid_spec = pltpu.PrefetchScalarGridSpec(
        num_scalar_prefetch=2,
        grid=(NBLK,),
        in_specs=[
            pl.BlockSpec((BT, D_MODEL), lambda g, be, bv: (g, 0)),
            pl.BlockSpec((1, D_MODEL, HID), lambda g, be, bv: (be[g], 0, 0)),
            pl.BlockSpec((1, D_MODEL, HID), lambda g, be, bv: (be[g], 0, 0)),
            pl.BlockSpec((1, HID, D_MODEL), lambda g, be, bv: (be[g], 0, 0)),
            pl.BlockSpec((1, 1, HID), lambda g, be, bv: (be[g], 0, 0)),
            pl.BlockSpec((1, 1, HID), lambda g, be, bv: (be[g], 0, 0)),
            pl.BlockSpec((1, 1, D_MODEL), lambda g, be, bv: (be[g], 0, 0)),
        ],
        out_specs=pl.BlockSpec((BT, D_MODEL), lambda g, be, bv: (g, 0)),
    )
    slots = pl.pallas_call(
        _ffn_body,
        grid_spec=grid_spec,
        out_shape=jax.ShapeDtypeStruct((GP, D_MODEL), jnp.float32),
    )(be.reshape(NBLK), bv.reshape(NBLK), xs,
      fc1_w, fc2_w, fc3_w,
      fc1_b.reshape(N_EXP, 1, HID), fc2_b.reshape(N_EXP, 1, HID),
      fc3_b.reshape(N_EXP, 1, D_MODEL))

    g0e = jnp.broadcast_to(gates[:, 0:1], (SEQ, 16))
    g1e = jnp.broadcast_to(gates[:, 1:2], (SEQ, 16))
    y = _sc_combine(slots, pos.T, g0e, g1e)
    return y.reshape(1, SEQ, D_MODEL)


# speedup vs baseline: 1.6723x; 1.0707x over previous
import dataclasses
import functools
import jax
import jax.numpy as jnp
from jax import lax
from jax.experimental import pallas as pl
from jax.experimental.pallas import tpu as pltpu
from jax.experimental.pallas import tpu_sc as plsc

D_MODEL = 768
N_EXP = 8
TOPK = 2
HID = 2048
SEQ = 2048
BT = 256
NBLK = (TOPK * SEQ) // BT + N_EXP
GP = NBLK * BT
NW = 32
CT = 64
_CS = 256


def _router_body(logits_ref, gates_ref, pos_ref, be_ref, bv_ref):
    logits = logits_ref[...]
    ei = jax.lax.broadcasted_iota(jnp.int32, (SEQ, N_EXP), 1)
    m0 = jnp.max(logits, axis=1, keepdims=True)
    e0 = jnp.min(jnp.where(logits == m0, ei, N_EXP), axis=1, keepdims=True)
    l2 = jnp.where(ei == e0, -jnp.inf, logits)
    m1 = jnp.max(l2, axis=1, keepdims=True)
    e1 = jnp.min(jnp.where(l2 == m1, ei, N_EXP), axis=1, keepdims=True)
    gates_ref[:, 0:1] = jax.nn.sigmoid(m0 - m1)
    gates_ref[:, 1:2] = jax.nn.sigmoid(m1 - m0)

    onehot0 = (ei == e0).astype(jnp.float32)
    onehot1 = (ei == e1).astype(jnp.float32)
    combined = (onehot0 + onehot1).astype(jnp.bfloat16)

    ri = jax.lax.broadcasted_iota(jnp.int32, (_CS, _CS), 0)
    ci = jax.lax.broadcasted_iota(jnp.int32, (_CS, _CS), 1)
    ltri = (ci < ri).astype(jnp.bfloat16)
    carry = jnp.zeros((1, N_EXP), jnp.float32)
    rank0_parts, rank1_parts = [], []
    for c in range(SEQ // _CS):
        sl = slice(c * _CS, (c + 1) * _CS)
        chunk = combined[sl, :]
        excl = jax.lax.dot_general(ltri, chunk, (((1,), (0,)), ((), ())),
                                   preferred_element_type=jnp.float32) + carry
        chunk_f = chunk.astype(jnp.float32)
        carry = excl[_CS - 1:_CS, :] + chunk_f[_CS - 1:_CS, :]
        rank0_parts.append(jnp.sum(excl * onehot0[sl, :], axis=1, keepdims=True))
        rank1_parts.append(jnp.sum((excl + onehot0[sl, :]) * onehot1[sl, :],
                                   axis=1, keepdims=True))
    counts = carry
    rank0 = jnp.concatenate(rank0_parts, axis=0)
    rank1 = jnp.concatenate(rank1_parts, axis=0)

    counts_i = counts.astype(jnp.int32)
    nb = (counts_i + BT - 1) // BT
    nb_f = nb.astype(jnp.bfloat16)
    u8r = jax.lax.broadcasted_iota(jnp.int32, (N_EXP, N_EXP), 0)
    u8c = jax.lax.broadcasted_iota(jnp.int32, (N_EXP, N_EXP), 1)
    utri = (u8r <= u8c).astype(jnp.bfloat16)
    nb_cum = jax.lax.dot_general(nb_f, utri, (((1,), (0,)), ((), ())),
                                 preferred_element_type=jnp.float32)
    pad_start = (BT * (nb_cum - nb.astype(jnp.float32)))

    p0 = jnp.sum(pad_start * onehot0, axis=1, keepdims=True) + rank0
    p1 = jnp.sum(pad_start * onehot1, axis=1, keepdims=True) + rank1
    pos_ref[:, 0:1] = p0.astype(jnp.int32)
    pos_ref[:, 1:2] = p1.astype(jnp.int32)

    gi = jax.lax.broadcasted_iota(jnp.int32, (N_EXP, NBLK), 1).astype(jnp.float32)
    nb_cum_t = jnp.transpose(nb_cum)
    bige = jnp.sum((gi >= nb_cum_t).astype(jnp.int32), axis=0, keepdims=True)
    be_ref[...] = jnp.minimum(bige, N_EXP - 1)
    nb_tot = nb_cum[:, N_EXP - 1:N_EXP]
    gi1 = jax.lax.broadcasted_iota(jnp.int32, (1, NBLK), 1).astype(jnp.float32)
    bv_ref[...] = (gi1 < nb_tot).astype(jnp.int32)


def _sc_dispatch(x2, pos_slot):
    mesh = plsc.VectorSubcoreMesh(core_axis_name="c", subcore_axis_name="s")
    b_per_w = GP // NW
    CH = 64
    NSL = TOPK * SEQ
    cp = pltpu.CompilerParams()
    if "needs_layout_passes" in pltpu.CompilerParams.__dataclass_fields__:
        cp = dataclasses.replace(cp, needs_layout_passes=False)

    @functools.partial(
        pl.kernel, mesh=mesh, compiler_params=cp,
        out_type=jax.ShapeDtypeStruct((GP, D_MODEL), jnp.float32),
        scratch_types=[
            pltpu.VMEM((NSL,), jnp.int32),
            pltpu.VMEM((GP,), jnp.int32),
            pltpu.VMEM((CH,), jnp.int32),
            pltpu.VMEM((CH,), jnp.int32),
            pltpu.VMEM((CH, D_MODEL), jnp.float32),
            pltpu.VMEM((CH, D_MODEL), jnp.float32),
            pltpu.VMEM_SHARED((GP,), jnp.int32),
            pltpu.SemaphoreType.DMA,
            pltpu.SemaphoreType.DMA,
        ],
    )
    def k(x_hbm, pos_hbm, out_hbm, pos_v, tok_img, iv0, iv1, rv0, rv1,
          tok_sh, s0, s1):
        cid = lax.axis_index("c")
        sid = lax.axis_index("s")
        wid = sid * 2 + cid

        @pl.when(sid == 0)
        def _build():
            pltpu.sync_copy(pos_hbm, pos_v)

            @pl.loop(0, GP // 16)
            def _(j):
                fill = (j * 16 + lax.iota(jnp.int32, 16)) & (SEQ - 1)
                tok_img[pl.ds(j * 16, 16)] = fill

            @pl.loop(0, NSL // 16)
            def _(j):
                idx = pos_v[pl.ds(j * 16, 16)]
                tok = (j * 16 + lax.iota(jnp.int32, 16)) // TOPK
                plsc.store_scatter(tok_img, [idx], tok)

            pltpu.sync_copy(tok_img, tok_sh)

        plsc.subcore_barrier()

        base = wid * b_per_w
        pltpu.sync_copy(tok_sh.at[pl.ds(base, CH)], iv0)
        g0 = pltpu.async_copy(x_hbm.at[iv0], rv0, s0)
        pltpu.sync_copy(tok_sh.at[pl.ds(base + CH, CH)], iv1)
        g1 = pltpu.async_copy(x_hbm.at[iv1], rv1, s1)
        g0.wait()
        pltpu.sync_copy(rv0, out_hbm.at[pl.ds(base, CH)])
        pltpu.sync_copy(tok_sh.at[pl.ds(base + 2 * CH, CH)], iv0)
        g2 = pltpu.async_copy(x_hbm.at[iv0], rv0, s0)
        g1.wait()
        pltpu.sync_copy(rv1, out_hbm.at[pl.ds(base + CH, CH)])
        g2.wait()
        pltpu.sync_copy(rv0, out_hbm.at[pl.ds(base + 2 * CH, CH)])

    return k(x2, pos_slot)


def _sc_combine(slots, pos2, g0e, g1e):
    mesh = plsc.VectorSubcoreMesh(core_axis_name="c", subcore_axis_name="s")
    t_per_w = SEQ // NW

    @functools.partial(
        pl.kernel, mesh=mesh,
        out_type=jax.ShapeDtypeStruct((SEQ, D_MODEL), jnp.float32),
        scratch_types=[
            pltpu.VMEM((CT,), jnp.int32),
            pltpu.VMEM((CT,), jnp.int32),
            pltpu.VMEM((CT, D_MODEL), jnp.float32),
            pltpu.VMEM((CT, D_MODEL), jnp.float32),
            pltpu.VMEM((CT, 16), jnp.float32),
            pltpu.VMEM((CT, 16), jnp.float32),
            pltpu.SemaphoreType.DMA,
            pltpu.SemaphoreType.DMA,
        ],
    )
    def k(slots_hbm, pos_hbm, g0_hbm, g1_hbm, y_hbm,
          i0_v, i1_v, r0_v, r1_v, gv0, gv1, s0, s1):
        wid = lax.axis_index("s") * 2 + lax.axis_index("c")
        base = wid * t_per_w
        for c in range(t_per_w // CT):
            off = base + c * CT
            pltpu.sync_copy(pos_hbm.at[0, pl.ds(off, CT)], i0_v)
            pltpu.sync_copy(pos_hbm.at[1, pl.ds(off, CT)], i1_v)
            cp0 = pltpu.async_copy(slots_hbm.at[i0_v], r0_v, s0)
            cp1 = pltpu.async_copy(slots_hbm.at[i1_v], r1_v, s1)
            pltpu.sync_copy(g0_hbm.at[pl.ds(off, CT)], gv0)
            pltpu.sync_copy(g1_hbm.at[pl.ds(off, CT)], gv1)
            cp0.wait()
            cp1.wait()

            @pl.loop(0, CT)
            def _(i):
                a = gv0[i, :]
                b = gv1[i, :]
                for j in range(D_MODEL // 16):
                    sl = pl.ds(j * 16, 16)
                    r0_v[i, sl] = r0_v[i, sl] * a + r1_v[i, sl] * b

            pltpu.sync_copy(r0_v, y_hbm.at[pl.ds(off, CT)])

    return k(slots, pos2, g0e, g1e)


def _ffn_body(be_ref, bv_ref, xs_ref, w1_ref, w2_ref, w3_ref,
              b1_ref, b2_ref, b3_ref, out_ref):
    g = pl.program_id(0)

    @pl.when(bv_ref[g] == 1)
    def _compute():
        xgb = xs_ref[...].astype(jnp.bfloat16)
        w1 = w1_ref[0].astype(jnp.bfloat16)
        w2 = w2_ref[0].astype(jnp.bfloat16)
        w3 = w3_ref[0].astype(jnp.bfloat16)
        h1 = jax.lax.dot_general(xgb, w1, (((1,), (0,)), ((), ())),
                                 preferred_element_type=jnp.float32) + b1_ref[0]
        h2 = jax.lax.dot_general(xgb, w2, (((1,), (0,)), ((), ())),
                                 preferred_element_type=jnp.float32) + b2_ref[0]
        hs = (h1 * (h2 * jax.nn.sigmoid(h2))).astype(jnp.bfloat16)
        out_ref[...] = jax.lax.dot_general(
            hs, w3, (((1,), (0,)), ((), ())),
            preferred_element_type=jnp.float32) + b3_ref[0]


def kernel(x, router_w, router_b, fc1_w, fc1_b, fc2_w, fc2_b, fc3_w, fc3_b):
    x2 = x.reshape(SEQ, D_MODEL)

    logits = (jnp.einsum('btd,de->bte', x, router_w) + router_b).reshape(SEQ, N_EXP)

    gates, pos, be, bv = pl.pallas_call(
        _router_body,
        out_shape=(jax.ShapeDtypeStruct((SEQ, TOPK), jnp.float32),
                   jax.ShapeDtypeStruct((SEQ, TOPK), jnp.int32),
                   jax.ShapeDtypeStruct((1, NBLK), jnp.int32),
                   jax.ShapeDtypeStruct((1, NBLK), jnp.int32)),
    )(logits)

    xs = _sc_dispatch(x2, pos.reshape(TOPK * SEQ))

    grid_spec = pltpu.PrefetchScalarGridSpec(
        num_scalar_prefetch=2,
        grid=(NBLK,),
        in_specs=[
            pl.BlockSpec((BT, D_MODEL), lambda g, be, bv: (g, 0)),
            pl.BlockSpec((1, D_MODEL, HID), lambda g, be, bv: (be[g], 0, 0)),
            pl.BlockSpec((1, D_MODEL, HID), lambda g, be, bv: (be[g], 0, 0)),
            pl.BlockSpec((1, HID, D_MODEL), lambda g, be, bv: (be[g], 0, 0)),
            pl.BlockSpec((1, 1, HID), lambda g, be, bv: (be[g], 0, 0)),
            pl.BlockSpec((1, 1, HID), lambda g, be, bv: (be[g], 0, 0)),
            pl.BlockSpec((1, 1, D_MODEL), lambda g, be, bv: (be[g], 0, 0)),
        ],
        out_specs=pl.BlockSpec((BT, D_MODEL), lambda g, be, bv: (g, 0)),
    )
    slots = pl.pallas_call(
        _ffn_body,
        grid_spec=grid_spec,
        out_shape=jax.ShapeDtypeStruct((GP, D_MODEL), jnp.float32),
    )(be.reshape(NBLK), bv.reshape(NBLK), xs,
      fc1_w, fc2_w, fc3_w,
      fc1_b.reshape(N_EXP, 1, HID), fc2_b.reshape(N_EXP, 1, HID),
      fc3_b.reshape(N_EXP, 1, D_MODEL))

    g0e = jnp.broadcast_to(gates[:, 0:1], (SEQ, 16))
    g1e = jnp.broadcast_to(gates[:, 1:2], (SEQ, 16))
    y = _sc_combine(slots, pos.T, g0e, g1e)
    return y.reshape(1, SEQ, D_MODEL)
